# baseline (speedup 1.0000x reference)
import functools

import jax
import jax.numpy as jnp
from jax.experimental import pallas as pl
from jax.experimental.pallas import tpu as pltpu


def _body(oq_ref, cin_ref, r_ref, cbmta_ref, cbmt_ref, od_ref, op_ref,
          cs_ref, cp_ref, *, m_sub, kc, rb1, g1, rb2, g2):
    f32, bf16 = jnp.float32, jnp.bfloat16
    i = pl.program_id(0)

    @pl.when(i < g1)
    def _prep():
        x16 = cin_ref[pl.ds(i * rb1, rb1), :]
        rot = jnp.dot(x16, r_ref[...], preferred_element_type=f32)
        rot16 = rot.astype(bf16)
        cs_ref[pl.ds(i * rb1, rb1), :] = rot16
        rot_aug = jnp.concatenate(
            [rot16, jnp.ones((rb1, 1), bf16)], axis=1)
        dims_t = (((1,), (1,)), ((), ()))
        dist_t = jax.lax.dot_general(cbmta_ref[...], rot_aug, dims_t,
                                     preferred_element_type=f32)
        d3 = dist_t.reshape(m_sub, kc, rb1)
        min3 = jnp.min(d3, axis=1, keepdims=True)
        oh_t = (d3 == min3).astype(bf16).reshape(m_sub * kc, rb1)
        dims_c = (((0,), (0,)), ((), ()))
        qnt = jax.lax.dot_general(oh_t, cbmt_ref[...], dims_c,
                                  preferred_element_type=f32)
        cp_ref[pl.ds(i * rb1, rb1), :] = qnt.astype(bf16)

    @pl.when(i >= g1)
    def _loss():
        li = i - g1
        oq16 = oq_ref[pl.ds(li * rb2, rb2), :]
        rq16 = jnp.dot(oq16, r_ref[...],
                       preferred_element_type=f32).astype(bf16)
        dims = (((1,), (1,)), ((), ()))

        t = jax.lax.dot_general(oq16, cin_ref[...], dims,
                                preferred_element_type=f32)
        mt = jnp.max(t, axis=1, keepdims=True)
        et = jnp.exp(t - mt)
        st = jnp.sum(et, axis=1, keepdims=True)

        s_d = jax.lax.dot_general(rq16, cs_ref[...], dims,
                                  preferred_element_type=f32)
        s_p = jax.lax.dot_general(rq16, cp_ref[...], dims,
                                  preferred_element_type=f32)
        mp = jnp.max(s_p, axis=1, keepdims=True)
        es_d = jnp.exp(s_d - mt)
        es_p = jnp.exp(s_p - mp)
        ss_d = jnp.sum(es_d, axis=1, keepdims=True)
        ss_p = jnp.sum(es_p, axis=1, keepdims=True)
        num_d = jnp.sum(et * jnp.log(es_d + 1e-6 * ss_d),
                        axis=1, keepdims=True)
        num_p = jnp.sum(et * jnp.log(es_p + 1e-6 * ss_p),
                        axis=1, keepdims=True)
        dense_part = jnp.sum(num_d / st - jnp.log(ss_d))
        pq_part = jnp.sum(num_p / st - jnp.log(ss_p))
        od_ref[pl.ds(li, 1), :, :] = jnp.full((1, 8, 128), dense_part, f32)
        op_ref[pl.ds(li, 1), :, :] = jnp.full((1, 8, 128), pq_part, f32)


def kernel(query_token_ids, query_attention_mask, doc_token_ids,
           doc_attention_mask, neg_token_ids, neg_attention_mask,
           origin_q_emb, origin_d_emb, origin_n_emb, doc_ids, neg_ids,
           R, codebook):
    f32, bf16 = jnp.float32, jnp.bfloat16
    b, emb = origin_q_emb.shape
    m_sub, kc, d_sub = codebook.shape
    mk = m_sub * kc
    n2 = 2 * b

    eye = jnp.eye(m_sub, dtype=codebook.dtype)
    cbmt = (eye[:, :, None, None] * codebook[:, None, :, :]) \
        .transpose(0, 2, 1, 3).reshape(mk, emb)
    n2col = jnp.sum(codebook * codebook, axis=-1).reshape(mk, 1)
    cbmta = jnp.concatenate([-2.0 * cbmt, n2col], axis=1)

    c_in16 = jnp.concatenate([origin_d_emb, origin_n_emb],
                             axis=0).astype(bf16)
    oq16 = origin_q_emb.astype(bf16)
    r16 = R.astype(bf16)
    cbmt16 = cbmt.astype(bf16)
    cbmta16 = cbmta.astype(bf16)

    rb1 = min(128, n2)
    g1 = n2 // rb1
    rb2 = min(256, b)
    g2 = b // rb2

    full = lambda shape: pl.BlockSpec(shape, lambda i: tuple(0 for _ in shape))
    partials = pl.pallas_call(
        functools.partial(_body, m_sub=m_sub, kc=kc,
                          rb1=rb1, g1=g1, rb2=rb2, g2=g2),
        grid=(g1 + g2,),
        in_specs=[
            full((b, emb)),
            full((n2, emb)),
            full((emb, emb)),
            full((mk, emb + 1)),
            full((mk, emb)),
        ],
        out_specs=[
            full((g2, 8, 128)),
            full((g2, 8, 128)),
        ],
        out_shape=[
            jax.ShapeDtypeStruct((g2, 8, 128), f32),
            jax.ShapeDtypeStruct((g2, 8, 128), f32),
        ],
        scratch_shapes=[
            pltpu.VMEM((n2, emb), bf16),
            pltpu.VMEM((n2, emb), bf16),
        ],
        compiler_params=pltpu.CompilerParams(
            dimension_semantics=("arbitrary",)),
    )(oq16, c_in16, r16, cbmta16, cbmt16)

    dense_loss = -jnp.sum(partials[0][:, 0, 0]) / b
    pq_loss = -jnp.sum(partials[1][:, 0, 0]) / b
    ivf_loss = jnp.asarray(0.0, dtype=f32)
    return (dense_loss, ivf_loss, pq_loss)


# speedup vs baseline: 1.1760x; 1.1760x over previous
import functools

import jax
import jax.numpy as jnp
from jax.experimental import pallas as pl
from jax.experimental.pallas import tpu as pltpu


def _body(oq_ref, cin_ref, r_ref, cbmta_ref, cbmt_ref, od_ref, op_ref,
          cs_ref, cp_ref, *, m_sub, kc, rb1, g1, rb2, g2):
    f32, bf16 = jnp.float32, jnp.bfloat16
    i = pl.program_id(0)

    @pl.when(i < g1)
    def _prep():
        x16 = cin_ref[pl.ds(i * rb1, rb1), :]
        rot = jnp.dot(x16, r_ref[...], preferred_element_type=f32)
        rot16 = rot.astype(bf16)
        cs_ref[pl.ds(i * rb1, rb1), :] = rot16
        rot_aug = jnp.concatenate(
            [rot16, jnp.ones((rb1, 1), bf16)], axis=1)
        dims_t = (((1,), (1,)), ((), ()))
        dist_t = jax.lax.dot_general(cbmta_ref[...], rot_aug, dims_t,
                                     preferred_element_type=f32)
        d3 = dist_t.reshape(m_sub, kc, rb1)
        min3 = jnp.min(d3, axis=1, keepdims=True)
        oh_t = (d3 == min3).astype(bf16).reshape(m_sub * kc, rb1)
        dims_c = (((0,), (0,)), ((), ()))
        qnt = jax.lax.dot_general(oh_t, cbmt_ref[...], dims_c,
                                  preferred_element_type=f32)
        cp_ref[pl.ds(i * rb1, rb1), :] = qnt.astype(bf16)

    @pl.when(i >= g1)
    def _loss():
        li = i - g1
        oq16 = oq_ref[pl.ds(li * rb2, rb2), :]
        rq16 = jnp.dot(oq16, r_ref[...],
                       preferred_element_type=f32).astype(bf16)
        dims = (((1,), (1,)), ((), ()))

        t = jax.lax.dot_general(oq16, cin_ref[...], dims,
                                preferred_element_type=f32)
        mt = jnp.max(t, axis=1, keepdims=True)
        et = jnp.exp(t - mt)
        st = jnp.sum(et, axis=1, keepdims=True)

        s_d = jax.lax.dot_general(rq16, cs_ref[...], dims,
                                  preferred_element_type=f32)
        s_p = jax.lax.dot_general(rq16, cp_ref[...], dims,
                                  preferred_element_type=f32)
        mp = jnp.max(s_p, axis=1, keepdims=True)
        es_d = jnp.exp(s_d - mt)
        es_p = jnp.exp(s_p - mp)
        ss_d = jnp.sum(es_d, axis=1, keepdims=True)
        ss_p = jnp.sum(es_p, axis=1, keepdims=True)
        num_d = jnp.sum(et * jnp.log(es_d + 1e-6 * ss_d),
                        axis=1, keepdims=True)
        num_p = jnp.sum(et * jnp.log(es_p + 1e-6 * ss_p),
                        axis=1, keepdims=True)
        dense_part = jnp.sum(num_d / st - jnp.log(ss_d))
        pq_part = jnp.sum(num_p / st - jnp.log(ss_p))
        od_ref[pl.ds(li, 1), :, :] = jnp.full((1, 8, 128), dense_part, f32)
        op_ref[pl.ds(li, 1), :, :] = jnp.full((1, 8, 128), pq_part, f32)


def kernel(query_token_ids, query_attention_mask, doc_token_ids,
           doc_attention_mask, neg_token_ids, neg_attention_mask,
           origin_q_emb, origin_d_emb, origin_n_emb, doc_ids, neg_ids,
           R, codebook):
    f32, bf16 = jnp.float32, jnp.bfloat16
    b, emb = origin_q_emb.shape
    m_sub, kc, d_sub = codebook.shape
    mk = m_sub * kc
    n2 = 2 * b

    eye = jnp.eye(m_sub, dtype=codebook.dtype)
    cbmt = (eye[:, :, None, None] * codebook[:, None, :, :]) \
        .transpose(0, 2, 1, 3).reshape(mk, emb)
    n2col = jnp.sum(codebook * codebook, axis=-1).reshape(mk, 1)
    cbmta = jnp.concatenate([-2.0 * cbmt, n2col], axis=1)

    c_in16 = jnp.concatenate([origin_d_emb, origin_n_emb],
                             axis=0).astype(bf16)
    oq16 = origin_q_emb.astype(bf16)
    r16 = R.astype(bf16)
    cbmt16 = cbmt.astype(bf16)
    cbmta16 = cbmta.astype(bf16)

    rb1 = min(256, n2)
    g1 = n2 // rb1
    rb2 = min(256, b)
    g2 = b // rb2

    full = lambda shape: pl.BlockSpec(shape, lambda i: tuple(0 for _ in shape))
    partials = pl.pallas_call(
        functools.partial(_body, m_sub=m_sub, kc=kc,
                          rb1=rb1, g1=g1, rb2=rb2, g2=g2),
        grid=(g1 + g2,),
        in_specs=[
            full((b, emb)),
            full((n2, emb)),
            full((emb, emb)),
            full((mk, emb + 1)),
            full((mk, emb)),
        ],
        out_specs=[
            full((g2, 8, 128)),
            full((g2, 8, 128)),
        ],
        out_shape=[
            jax.ShapeDtypeStruct((g2, 8, 128), f32),
            jax.ShapeDtypeStruct((g2, 8, 128), f32),
        ],
        scratch_shapes=[
            pltpu.VMEM((n2, emb), bf16),
            pltpu.VMEM((n2, emb), bf16),
        ],
        compiler_params=pltpu.CompilerParams(
            dimension_semantics=("arbitrary",)),
    )(oq16, c_in16, r16, cbmta16, cbmt16)

    dense_loss = -jnp.sum(partials[0][:, 0, 0]) / b
    pq_loss = -jnp.sum(partials[1][:, 0, 0]) / b
    ivf_loss = jnp.asarray(0.0, dtype=f32)
    return (dense_loss, ivf_loss, pq_loss)
